# baseline (speedup 1.0000x reference)
import jax
import jax.numpy as jnp
from jax.experimental import pallas as pl

N_SUB = 1000
BLOCK = 8192
CHUNK = 512
assert BLOCK % CHUNK == 0, "chunk loop must tile the block exactly"


def _bdot(a, b, dims=None):
    if dims is None:
        return jnp.dot(a.astype(jnp.bfloat16), b, preferred_element_type=jnp.float32)
    return jax.lax.dot_general(a.astype(jnp.bfloat16), b, dims,
                               preferred_element_type=jnp.float32)


def _fused_kernel(xa_ref, xb_ref, w1_ref, b1_ref, w2t_ref, b2r_ref,
                  fsub_ref, adj_ref, wg1_ref, wmu_ref, wlogvar_ref, epst_ref,
                  oxt_ref, oloss_ref):
    f32 = jnp.float32
    bf16 = jnp.bfloat16
    pid = pl.program_id(0)
    w1 = w1_ref[...].astype(bf16)
    w2t = w2t_ref[...].astype(bf16)
    b1 = b1_ref[...]

    @pl.when(pid == 0)
    def _gae():
        n = float(N_SUB)

        h0 = _bdot(fsub_ref[...], w1)
        h0 = jnp.maximum(h0 + b1, 0.0)
        x1 = _bdot(h0, w2t, (((1,), (1,)), ((), ()))) + b2r_ref[...]

        row = jax.lax.broadcasted_iota(jnp.int32, (N_SUB, 1), 0)
        col = jax.lax.broadcasted_iota(jnp.int32, (1, N_SUB), 1)

        adj = adj_ref[...]
        eye = jnp.where(row == col, 1.0, 0.0)
        a_hat = adj + eye

        d_row = jnp.sum(a_hat, axis=1, keepdims=True)
        dinv_row = jax.lax.rsqrt(d_row)
        dinv_col = jnp.transpose(dinv_row, (1, 0))
        adj_norm = (a_hat * dinv_row * dinv_col).astype(bf16)

        h = jnp.maximum(_bdot(adj_norm, _bdot(x1, wg1_ref[...].astype(bf16)).astype(bf16)), 0.0)
        hb = h.astype(bf16)
        mu = _bdot(adj_norm, _bdot(hb, wmu_ref[...].astype(bf16)).astype(bf16))
        logvar = _bdot(adj_norm, _bdot(hb, wlogvar_ref[...].astype(bf16)).astype(bf16))
        std = jnp.exp(logvar)
        eps = jnp.transpose(epst_ref[...], (1, 0))
        z = mu + eps * std

        zb = z.astype(bf16)
        preds = jax.lax.dot_general(zb, zb, (((1,), (1,)), ((), ())),
                                    preferred_element_type=f32)

        adj_sum = jnp.sum(d_row) - n
        nn = n * n
        norm = nn / ((nn - adj_sum) * 2.0)
        pos_weight = (nn - adj_sum) / adj_sum

        sp = jnp.log1p(jnp.exp(-jnp.abs(preds))) + jnp.maximum(-preds, 0.0)
        bce = jnp.where(a_hat > 0.0, pos_weight * sp, preds + sp)
        cost = norm * (jnp.sum(bce) / nn)

        term = 1.0 + 2.0 * logvar - mu * mu - std * std
        kld = (-0.5 / n) * (jnp.sum(term) / n)

        oloss_ref[...] = jnp.reshape(cost + kld, (1, 1))

    @pl.when(pid != 0)
    def _mlp():
        b2 = jnp.transpose(b2r_ref[...], (1, 0))
        half = BLOCK // 2
        for c in range(BLOCK // CHUNK):
            base = c * CHUNK
            if base < half:
                src = xa_ref[pl.ds(base, CHUNK), :]
            else:
                src = xb_ref[pl.ds(base - half, CHUNK), :]
            h = _bdot(src, w1)
            h = jnp.maximum(h + b1, 0.0)
            ot = _bdot(w2t, h.astype(bf16), (((1,), (1,)), ((), ())))
            oxt_ref[:, pl.ds(base, CHUNK)] = ot + b2


def kernel(feature_sub_graph, adj, ori_adj, ori_feature, W1, b1, W2, b2,
           Wg1, Wmu, Wlogvar, eps):
    n_ori, d_feat = ori_feature.shape
    d_hid = W1.shape[1]
    d_lat = W2.shape[1]
    b1r = b1.reshape(1, d_hid)
    b2r = b2.reshape(1, d_lat)
    W2t = jnp.transpose(W2, (1, 0))
    epst = jnp.transpose(eps, (1, 0))

    num_blocks = (n_ori + BLOCK - 1) // BLOCK
    last_half = (n_ori - 1) // (BLOCK // 2)
    mlp_idx_a = lambda i: (jnp.clip(2 * (i - 1), 0, last_half), 0)
    mlp_idx_b = lambda i: (jnp.clip(2 * (i - 1) + 1, 1, last_half), 0)
    const_idx = lambda i: (0, 0)

    xt, loss = pl.pallas_call(
        _fused_kernel,
        grid=(num_blocks + 1,),
        in_specs=[
            pl.BlockSpec((BLOCK // 2, d_feat), mlp_idx_a),
            pl.BlockSpec((BLOCK // 2, d_feat), mlp_idx_b),
            pl.BlockSpec((d_feat, d_hid), const_idx),
            pl.BlockSpec((1, d_hid), const_idx),
            pl.BlockSpec((d_lat, d_hid), const_idx),
            pl.BlockSpec((1, d_lat), const_idx),
            pl.BlockSpec((N_SUB, d_feat), const_idx),
            pl.BlockSpec((N_SUB, N_SUB), const_idx),
            pl.BlockSpec((d_lat, d_lat), const_idx),
            pl.BlockSpec((d_lat, d_lat), const_idx),
            pl.BlockSpec((d_lat, d_lat), const_idx),
            pl.BlockSpec((d_lat, N_SUB), const_idx),
        ],
        out_specs=[
            pl.BlockSpec((d_lat, BLOCK), lambda i: (0, jnp.maximum(i - 1, 0))),
            pl.BlockSpec((1, 1), const_idx),
        ],
        out_shape=[
            jax.ShapeDtypeStruct((d_lat, n_ori), jnp.float32),
            jax.ShapeDtypeStruct((1, 1), jnp.float32),
        ],
    )(ori_feature, ori_feature, W1, b1r, W2t, b2r,
      feature_sub_graph, adj, Wg1, Wmu, Wlogvar, epst)

    return (jnp.transpose(xt, (1, 0)), loss[0, 0])


# speedup vs baseline: 1.0074x; 1.0074x over previous
import jax
import jax.numpy as jnp
from jax.experimental import pallas as pl

N_SUB = 1000
BLOCK = 8192
CHUNK = 512
assert BLOCK % CHUNK == 0, "chunk loop must tile the block exactly"


def _bdot(a, b, dims=None):
    if dims is None:
        return jnp.dot(a.astype(jnp.bfloat16), b, preferred_element_type=jnp.float32)
    return jax.lax.dot_general(a.astype(jnp.bfloat16), b, dims,
                               preferred_element_type=jnp.float32)


def _fused_kernel(x_ref, w1_ref, b1_ref, w2t_ref, b2r_ref,
                  fsub_ref, adj_ref, wg1_ref, wmu_ref, wlogvar_ref, epst_ref,
                  oxt_ref, oloss_ref):
    f32 = jnp.float32
    bf16 = jnp.bfloat16
    pid = pl.program_id(0)
    w1 = w1_ref[...].astype(bf16)
    w2t = w2t_ref[...].astype(bf16)
    b1 = b1_ref[...]

    @pl.when(pid == 0)
    def _gae():
        n = float(N_SUB)

        h0 = _bdot(fsub_ref[...], w1)
        h0 = jnp.maximum(h0 + b1, 0.0)
        x1 = _bdot(h0, w2t, (((1,), (1,)), ((), ()))) + b2r_ref[...]

        row = jax.lax.broadcasted_iota(jnp.int32, (N_SUB, 1), 0)
        col = jax.lax.broadcasted_iota(jnp.int32, (1, N_SUB), 1)

        adj = adj_ref[...]
        eye = jnp.where(row == col, 1.0, 0.0)
        a_hat = adj + eye

        d_row = jnp.sum(a_hat, axis=1, keepdims=True)
        dinv_row = jax.lax.rsqrt(d_row)
        dinv_col = jnp.transpose(dinv_row, (1, 0))
        adj_norm = (a_hat * dinv_row * dinv_col).astype(bf16)

        h = jnp.maximum(_bdot(adj_norm, _bdot(x1, wg1_ref[...].astype(bf16)).astype(bf16)), 0.0)
        hb = h.astype(bf16)
        mu = _bdot(adj_norm, _bdot(hb, wmu_ref[...].astype(bf16)).astype(bf16))
        logvar = _bdot(adj_norm, _bdot(hb, wlogvar_ref[...].astype(bf16)).astype(bf16))
        std = jnp.exp(logvar)
        eps = jnp.transpose(epst_ref[...], (1, 0))
        z = mu + eps * std

        zb = z.astype(bf16)
        preds = jax.lax.dot_general(zb, zb, (((1,), (1,)), ((), ())),
                                    preferred_element_type=f32)

        adj_sum = jnp.sum(d_row) - n
        nn = n * n
        norm = nn / ((nn - adj_sum) * 2.0)
        pos_weight = (nn - adj_sum) / adj_sum

        sp = jnp.log1p(jnp.exp(-jnp.abs(preds))) + jnp.maximum(-preds, 0.0)
        bce = jnp.where(a_hat > 0.0, pos_weight * sp, preds + sp)
        cost = norm * (jnp.sum(bce) / nn)

        term = 1.0 + 2.0 * logvar - mu * mu - std * std
        kld = (-0.5 / n) * (jnp.sum(term) / n)

        oloss_ref[...] = jnp.reshape(cost + kld, (1, 1))

    @pl.when(pid != 0)
    def _mlp():
        b2 = jnp.transpose(b2r_ref[...], (1, 0))
        for c in range(BLOCK // CHUNK):
            sl = pl.ds(c * CHUNK, CHUNK)
            h = _bdot(x_ref[sl, :], w1)
            h = jnp.maximum(h + b1, 0.0)
            ot = _bdot(w2t, h.astype(bf16), (((1,), (1,)), ((), ())))
            oxt_ref[:, sl] = ot + b2


def kernel(feature_sub_graph, adj, ori_adj, ori_feature, W1, b1, W2, b2,
           Wg1, Wmu, Wlogvar, eps):
    n_ori, d_feat = ori_feature.shape
    d_hid = W1.shape[1]
    d_lat = W2.shape[1]
    b1r = b1.reshape(1, d_hid)
    b2r = b2.reshape(1, d_lat)
    W2t = jnp.transpose(W2, (1, 0))
    epst = jnp.transpose(eps, (1, 0))

    num_blocks = (n_ori + BLOCK - 1) // BLOCK
    mlp_idx = lambda i: (jnp.maximum(i - 1, 0), 0)
    const_idx = lambda i: (0, 0)

    xt, loss = pl.pallas_call(
        _fused_kernel,
        grid=(num_blocks + 1,),
        in_specs=[
            pl.BlockSpec((BLOCK, d_feat), mlp_idx),
            pl.BlockSpec((d_feat, d_hid), const_idx),
            pl.BlockSpec((1, d_hid), const_idx),
            pl.BlockSpec((d_lat, d_hid), const_idx),
            pl.BlockSpec((1, d_lat), const_idx),
            pl.BlockSpec((N_SUB, d_feat), const_idx),
            pl.BlockSpec((N_SUB, N_SUB), const_idx),
            pl.BlockSpec((d_lat, d_lat), const_idx),
            pl.BlockSpec((d_lat, d_lat), const_idx),
            pl.BlockSpec((d_lat, d_lat), const_idx),
            pl.BlockSpec((d_lat, N_SUB), const_idx),
        ],
        out_specs=[
            pl.BlockSpec((d_lat, BLOCK), lambda i: (0, jnp.maximum(i - 1, 0))),
            pl.BlockSpec((1, 1), const_idx),
        ],
        out_shape=[
            jax.ShapeDtypeStruct((d_lat, n_ori), jnp.float32),
            jax.ShapeDtypeStruct((1, 1), jnp.float32),
        ],
    )(ori_feature, W1, b1r, W2t, b2r,
      feature_sub_graph, adj, Wg1, Wmu, Wlogvar, epst)

    return (jnp.transpose(xt, (1, 0)), loss[0, 0])
